# baseline (speedup 1.0000x reference)
import functools
import jax
import jax.numpy as jnp
from jax import lax
from jax.experimental import pallas as pl
from jax.experimental.pallas import tpu as pltpu
from jax.experimental.pallas import tpu_sc as plsc

_BLK = 512
_RB = 512
_SRC_ROWS = 16
_SC_FRAC = 2


def _read_kernel(a_ref, d_ref, acc_ref):
    i = pl.program_id(0)

    @pl.when(i == 0)
    def _():
        acc_ref[...] = jnp.zeros_like(acc_ref)

    acc_ref[...] += jnp.sum(a_ref[...], axis=0, keepdims=True)

    @pl.when(i == pl.num_programs(0) - 1)
    def _():
        d_ref[...] = (acc_ref[...] == 1.0).astype(jnp.float32)


def _write_kernel(inout_ref, ones_ref):
    del inout_ref
    ones_ref[...] = jnp.ones_like(ones_ref)


def _fix_kernel(flags_ref, d_ref, a_ref, inout_ref, out_ref,
                aij_s, aji_s, res_s, dj_s, di_s, sem_a, sem_b, sem_o,
                sem_dj, sem_di):
    del inout_ref
    t = flags_ref.shape[0]

    def body(r, carry):
        i = r // t
        j = r % t

        @pl.when((flags_ref[i] | flags_ref[j]) > 0)
        def _():
            cp_a = pltpu.make_async_copy(
                a_ref.at[pl.ds(i * _BLK, _BLK), pl.ds(j * _BLK, _BLK)],
                aij_s, sem_a)
            cp_b = pltpu.make_async_copy(
                a_ref.at[pl.ds(j * _BLK, _BLK), pl.ds(i * _BLK, _BLK)],
                aji_s, sem_b)
            cp_dj = pltpu.make_async_copy(
                d_ref.at[:, pl.ds(j * _BLK, _BLK)], dj_s, sem_dj)
            cp_di = pltpu.make_async_copy(
                d_ref.at[:, pl.ds(i * _BLK, _BLK)], di_s, sem_di)
            cp_a.start()
            cp_b.start()
            cp_dj.start()
            cp_di.start()
            cp_a.wait()
            cp_b.wait()
            cp_dj.wait()
            cp_di.wait()
            res_s[...] = (1.0 - aij_s[...] * dj_s[...]
                          - (aji_s[...] * di_s[...]).T)
            cp_o = pltpu.make_async_copy(
                res_s, out_ref.at[pl.ds(i * _BLK, _BLK), pl.ds(j * _BLK, _BLK)],
                sem_o)
            cp_o.start()
            cp_o.wait()

        return carry

    jax.lax.fori_loop(0, t * t, body, 0)


def kernel(modified_adj):
    n = modified_adj.shape[0]
    t = n // _BLK
    sc_rows = n // _SC_FRAC
    info = plsc.get_sparse_core_info()
    nw = info.num_cores * info.num_subcores
    rows_per_w = sc_rows // nw
    sc_iters = rows_per_w // _SRC_ROWS
    mesh = plsc.VectorSubcoreMesh(core_axis_name="c", subcore_axis_name="s")

    @functools.partial(
        pl.kernel,
        out_type=jax.ShapeDtypeStruct((n, n), jnp.float32),
        mesh=mesh,
        scratch_types=[
            pltpu.VMEM((_SRC_ROWS, n), jnp.float32),
            pltpu.SemaphoreType.DMA,
        ],
    )
    def sc_ones_top(src_hbm, out_hbm, buf_v, sem):
        wid = lax.axis_index("s") * info.num_cores + lax.axis_index("c")
        base = wid * rows_per_w
        pltpu.sync_copy(src_hbm, buf_v)

        def body(k, carry):
            pltpu.async_copy(
                buf_v, out_hbm.at[pl.ds(base + k * _SRC_ROWS, _SRC_ROWS), :],
                sem)
            return carry

        lax.fori_loop(0, sc_iters, body, 0)

        def drain(k, carry):
            pltpu.make_async_copy(
                buf_v, out_hbm.at[pl.ds(base + k * _SRC_ROWS, _SRC_ROWS), :],
                sem).wait()
            return carry

        lax.fori_loop(0, sc_iters, drain, 0)

    src = jnp.ones((_SRC_ROWS, n), jnp.float32)
    ones_top = sc_ones_top(src)

    d = pl.pallas_call(
        _read_kernel,
        grid=(n // _RB,),
        in_specs=[pl.BlockSpec((_RB, n), lambda i: (i, 0))],
        out_specs=pl.BlockSpec((1, n), lambda i: (0, 0)),
        out_shape=jax.ShapeDtypeStruct((1, n), jnp.float32),
        scratch_shapes=[pltpu.VMEM((1, n), jnp.float32)],
    )(modified_adj)

    bot_steps = (n - sc_rows) // _RB
    bot_off = sc_rows // _RB
    ones = pl.pallas_call(
        _write_kernel,
        grid=(bot_steps,),
        in_specs=[pl.BlockSpec(memory_space=pltpu.MemorySpace.HBM)],
        out_specs=pl.BlockSpec((_RB, n), lambda i: (i + bot_off, 0)),
        out_shape=jax.ShapeDtypeStruct((n, n), jnp.float32),
        input_output_aliases={0: 0},
    )(ones_top)

    flags = (jnp.max(d.reshape(t, _BLK), axis=1) > 0.0).astype(jnp.int32)

    out = pl.pallas_call(
        _fix_kernel,
        grid=(1,),
        in_specs=[
            pl.BlockSpec(memory_space=pltpu.MemorySpace.SMEM),
            pl.BlockSpec(memory_space=pltpu.MemorySpace.HBM),
            pl.BlockSpec(memory_space=pltpu.MemorySpace.HBM),
            pl.BlockSpec(memory_space=pltpu.MemorySpace.HBM),
        ],
        out_specs=pl.BlockSpec(memory_space=pltpu.MemorySpace.HBM),
        out_shape=jax.ShapeDtypeStruct((n, n), jnp.float32),
        input_output_aliases={3: 0},
        scratch_shapes=[
            pltpu.VMEM((_BLK, _BLK), jnp.float32),
            pltpu.VMEM((_BLK, _BLK), jnp.float32),
            pltpu.VMEM((_BLK, _BLK), jnp.float32),
            pltpu.VMEM((1, _BLK), jnp.float32),
            pltpu.VMEM((1, _BLK), jnp.float32),
            pltpu.SemaphoreType.DMA,
            pltpu.SemaphoreType.DMA,
            pltpu.SemaphoreType.DMA,
            pltpu.SemaphoreType.DMA,
            pltpu.SemaphoreType.DMA,
        ],
    )(flags, d, modified_adj, ones)
    return out


# speedup vs baseline: 1.6024x; 1.6024x over previous
import jax
import jax.numpy as jnp
from jax.experimental import pallas as pl
from jax.experimental.pallas import tpu as pltpu

_BLK = 512
_RB = 512


def _read_kernel(a_ref, d_ref, acc_ref):
    i = pl.program_id(0)

    @pl.when(i == 0)
    def _():
        acc_ref[...] = jnp.zeros_like(acc_ref)

    acc_ref[...] += jnp.sum(a_ref[...], axis=0, keepdims=True)

    @pl.when(i == pl.num_programs(0) - 1)
    def _():
        d_ref[...] = (acc_ref[...] == 1.0).astype(jnp.float32)


def _write_fix_kernel(d_ref, dhbm_ref, a_ref, out_ref,
                      ones_s, flags_s, aij_s, aji_s, res_s, dj_s, di_s,
                      sem_w, sem_a, sem_b, sem_o, sem_dj, sem_di):
    n = d_ref.shape[1]
    t = n // _BLK
    rsteps = n // _RB

    ones_s[...] = jnp.ones_like(ones_s)
    for k in range(rsteps):
        pltpu.make_async_copy(
            ones_s, out_ref.at[pl.ds(k * _RB, _RB), :], sem_w).start()

    for b in range(t):
        flags_s[b] = (jnp.max(d_ref[:, b * _BLK:(b + 1) * _BLK]) > 0.0
                      ).astype(jnp.int32)

    for k in range(rsteps):
        pltpu.make_async_copy(
            ones_s, out_ref.at[pl.ds(k * _RB, _RB), :], sem_w).wait()

    def body(r, carry):
        i = r // t
        j = r % t

        @pl.when((flags_s[i] | flags_s[j]) > 0)
        def _():
            cp_a = pltpu.make_async_copy(
                a_ref.at[pl.ds(i * _BLK, _BLK), pl.ds(j * _BLK, _BLK)],
                aij_s, sem_a)
            cp_b = pltpu.make_async_copy(
                a_ref.at[pl.ds(j * _BLK, _BLK), pl.ds(i * _BLK, _BLK)],
                aji_s, sem_b)
            cp_dj = pltpu.make_async_copy(
                dhbm_ref.at[:, pl.ds(j * _BLK, _BLK)], dj_s, sem_dj)
            cp_di = pltpu.make_async_copy(
                dhbm_ref.at[:, pl.ds(i * _BLK, _BLK)], di_s, sem_di)
            cp_a.start()
            cp_b.start()
            cp_dj.start()
            cp_di.start()
            cp_a.wait()
            cp_b.wait()
            cp_dj.wait()
            cp_di.wait()
            res_s[...] = (1.0 - aij_s[...] * dj_s[...]
                          - (aji_s[...] * di_s[...]).T)
            cp_o = pltpu.make_async_copy(
                res_s, out_ref.at[pl.ds(i * _BLK, _BLK), pl.ds(j * _BLK, _BLK)],
                sem_o)
            cp_o.start()
            cp_o.wait()

        return carry

    jax.lax.fori_loop(0, t * t, body, 0)


def kernel(modified_adj):
    n = modified_adj.shape[0]
    t = n // _BLK

    d = pl.pallas_call(
        _read_kernel,
        grid=(n // _RB,),
        in_specs=[pl.BlockSpec((_RB, n), lambda i: (i, 0))],
        out_specs=pl.BlockSpec((1, n), lambda i: (0, 0)),
        out_shape=jax.ShapeDtypeStruct((1, n), jnp.float32),
        scratch_shapes=[pltpu.VMEM((1, n), jnp.float32)],
    )(modified_adj)

    out = pl.pallas_call(
        _write_fix_kernel,
        grid=(1,),
        in_specs=[
            pl.BlockSpec((1, n), lambda g: (0, 0)),
            pl.BlockSpec(memory_space=pltpu.MemorySpace.HBM),
            pl.BlockSpec(memory_space=pltpu.MemorySpace.HBM),
        ],
        out_specs=pl.BlockSpec(memory_space=pltpu.MemorySpace.HBM),
        out_shape=jax.ShapeDtypeStruct((n, n), jnp.float32),
        scratch_shapes=[
            pltpu.VMEM((_RB, n), jnp.float32),
            pltpu.SMEM((8,), jnp.int32),
            pltpu.VMEM((_BLK, _BLK), jnp.float32),
            pltpu.VMEM((_BLK, _BLK), jnp.float32),
            pltpu.VMEM((_BLK, _BLK), jnp.float32),
            pltpu.VMEM((1, _BLK), jnp.float32),
            pltpu.VMEM((1, _BLK), jnp.float32),
            pltpu.SemaphoreType.DMA,
            pltpu.SemaphoreType.DMA,
            pltpu.SemaphoreType.DMA,
            pltpu.SemaphoreType.DMA,
            pltpu.SemaphoreType.DMA,
            pltpu.SemaphoreType.DMA,
        ],
    )(d, d, modified_adj)
    return out


# speedup vs baseline: 1.6583x; 1.0349x over previous
import jax
import jax.numpy as jnp
from jax.experimental import pallas as pl
from jax.experimental.pallas import tpu as pltpu

_BLK = 512
_RB = 512
_NBUF = 3


def _fused_kernel(a_ref, out_ref,
                  buf0, buf1, buf2, acc_s, dblk_s, flags_s,
                  aij_s, aji_s, res_s,
                  sem_r0, sem_r1, sem_r2, sem_w, sem_a, sem_b, sem_o):
    n = a_ref.shape[0]
    t = n // _BLK
    rsteps = n // _RB
    bufs = (buf0, buf1, buf2)
    sems = (sem_r0, sem_r1, sem_r2)

    def strip_copy(k, slot):
        return pltpu.make_async_copy(
            a_ref.at[pl.ds(k * _RB, _RB), :], bufs[slot], sems[slot])

    for k in range(min(_NBUF, rsteps)):
        strip_copy(k, k % _NBUF).start()
    for k in range(rsteps):
        slot = k % _NBUF
        strip_copy(k, slot).wait()
        s = jnp.sum(bufs[slot][...], axis=0, keepdims=True)
        if k == 0:
            acc_s[...] = s
        else:
            acc_s[...] += s
        if k + _NBUF < rsteps:
            strip_copy(k + _NBUF, slot).start()

    d_v = (acc_s[...] == 1.0).astype(jnp.float32)

    for b in range(t):
        blk = d_v[:, b * _BLK:(b + 1) * _BLK]
        dblk_s[b, :, :] = blk
        flags_s[b] = (jnp.max(blk) > 0.0).astype(jnp.int32)

    buf0[...] = jnp.ones_like(buf0)
    for k in range(rsteps):
        pltpu.make_async_copy(
            buf0, out_ref.at[pl.ds(k * _RB, _RB), :], sem_w).start()
    for k in range(rsteps):
        pltpu.make_async_copy(
            buf0, out_ref.at[pl.ds(k * _RB, _RB), :], sem_w).wait()

    def body(r, carry):
        i = r // t
        j = r % t

        @pl.when((flags_s[i] | flags_s[j]) > 0)
        def _():
            cp_a = pltpu.make_async_copy(
                a_ref.at[pl.ds(i * _BLK, _BLK), pl.ds(j * _BLK, _BLK)],
                aij_s, sem_a)
            cp_b = pltpu.make_async_copy(
                a_ref.at[pl.ds(j * _BLK, _BLK), pl.ds(i * _BLK, _BLK)],
                aji_s, sem_b)
            cp_a.start()
            cp_b.start()
            cp_a.wait()
            cp_b.wait()
            dj = dblk_s[j, 0, :]
            di = dblk_s[i, 0, :]
            res_s[...] = (1.0 - aij_s[...] * dj[None, :]
                          - (aji_s[...] * di[None, :]).T)
            cp_o = pltpu.make_async_copy(
                res_s, out_ref.at[pl.ds(i * _BLK, _BLK), pl.ds(j * _BLK, _BLK)],
                sem_o)
            cp_o.start()
            cp_o.wait()

        return carry

    jax.lax.fori_loop(0, t * t, body, 0)


def kernel(modified_adj):
    n = modified_adj.shape[0]

    out = pl.pallas_call(
        _fused_kernel,
        grid=(1,),
        in_specs=[pl.BlockSpec(memory_space=pltpu.MemorySpace.HBM)],
        out_specs=pl.BlockSpec(memory_space=pltpu.MemorySpace.HBM),
        out_shape=jax.ShapeDtypeStruct((n, n), jnp.float32),
        scratch_shapes=[
            pltpu.VMEM((_RB, n), jnp.float32),
            pltpu.VMEM((_RB, n), jnp.float32),
            pltpu.VMEM((_RB, n), jnp.float32),
            pltpu.VMEM((1, n), jnp.float32),
            pltpu.VMEM((8, 1, _BLK), jnp.float32),
            pltpu.SMEM((8,), jnp.int32),
            pltpu.VMEM((_BLK, _BLK), jnp.float32),
            pltpu.VMEM((_BLK, _BLK), jnp.float32),
            pltpu.VMEM((_BLK, _BLK), jnp.float32),
            pltpu.SemaphoreType.DMA,
            pltpu.SemaphoreType.DMA,
            pltpu.SemaphoreType.DMA,
            pltpu.SemaphoreType.DMA,
            pltpu.SemaphoreType.DMA,
            pltpu.SemaphoreType.DMA,
            pltpu.SemaphoreType.DMA,
        ],
    )(modified_adj)
    return out


# speedup vs baseline: 1.6794x; 1.0127x over previous
import jax
import jax.numpy as jnp
from jax.experimental import pallas as pl
from jax.experimental.pallas import tpu as pltpu

_BLK = 512
_RB = 256
_NBUF = 5


def _fused_kernel(a_ref, out_ref,
                  buf0, buf1, buf2, buf3, buf4, acc_s, dblk_s, flags_s,
                  aij_s, aji_s, res_s,
                  sem_r0, sem_r1, sem_r2, sem_r3, sem_r4,
                  sem_w, sem_a, sem_b, sem_o):
    n = a_ref.shape[0]
    t = n // _BLK
    rsteps = n // _RB
    bufs = (buf0, buf1, buf2, buf3, buf4)
    sems = (sem_r0, sem_r1, sem_r2, sem_r3, sem_r4)

    def strip_copy(k, slot):
        return pltpu.make_async_copy(
            a_ref.at[pl.ds(k * _RB, _RB), :], bufs[slot], sems[slot])

    for k in range(min(_NBUF, rsteps)):
        strip_copy(k, k % _NBUF).start()
    for k in range(rsteps):
        slot = k % _NBUF
        strip_copy(k, slot).wait()
        s = jnp.sum(bufs[slot][...], axis=0, keepdims=True)
        if k == 0:
            acc_s[...] = s
        else:
            acc_s[...] += s
        if k + _NBUF < rsteps:
            strip_copy(k + _NBUF, slot).start()

    d_v = (acc_s[...] == 1.0).astype(jnp.float32)

    for b in range(t):
        blk = d_v[:, b * _BLK:(b + 1) * _BLK]
        dblk_s[b, :, :] = blk
        flags_s[b] = (jnp.max(blk) > 0.0).astype(jnp.int32)

    buf0[...] = jnp.ones_like(buf0)
    for k in range(rsteps):
        pltpu.make_async_copy(
            buf0, out_ref.at[pl.ds(k * _RB, _RB), :], sem_w).start()
    for k in range(rsteps):
        pltpu.make_async_copy(
            buf0, out_ref.at[pl.ds(k * _RB, _RB), :], sem_w).wait()

    def body(r, carry):
        i = r // t
        j = r % t

        @pl.when((flags_s[i] | flags_s[j]) > 0)
        def _():
            cp_a = pltpu.make_async_copy(
                a_ref.at[pl.ds(i * _BLK, _BLK), pl.ds(j * _BLK, _BLK)],
                aij_s, sem_a)
            cp_b = pltpu.make_async_copy(
                a_ref.at[pl.ds(j * _BLK, _BLK), pl.ds(i * _BLK, _BLK)],
                aji_s, sem_b)
            cp_a.start()
            cp_b.start()
            cp_a.wait()
            cp_b.wait()
            dj = dblk_s[j, 0, :]
            di = dblk_s[i, 0, :]
            res_s[...] = (1.0 - aij_s[...] * dj[None, :]
                          - (aji_s[...] * di[None, :]).T)
            cp_o = pltpu.make_async_copy(
                res_s, out_ref.at[pl.ds(i * _BLK, _BLK), pl.ds(j * _BLK, _BLK)],
                sem_o)
            cp_o.start()
            cp_o.wait()

        return carry

    jax.lax.fori_loop(0, t * t, body, 0)


def kernel(modified_adj):
    n = modified_adj.shape[0]

    out = pl.pallas_call(
        _fused_kernel,
        grid=(1,),
        in_specs=[pl.BlockSpec(memory_space=pltpu.MemorySpace.HBM)],
        out_specs=pl.BlockSpec(memory_space=pltpu.MemorySpace.HBM),
        out_shape=jax.ShapeDtypeStruct((n, n), jnp.float32),
        scratch_shapes=[
            pltpu.VMEM((_RB, n), jnp.float32),
            pltpu.VMEM((_RB, n), jnp.float32),
            pltpu.VMEM((_RB, n), jnp.float32),
            pltpu.VMEM((_RB, n), jnp.float32),
            pltpu.VMEM((_RB, n), jnp.float32),
            pltpu.VMEM((1, n), jnp.float32),
            pltpu.VMEM((8, 1, _BLK), jnp.float32),
            pltpu.SMEM((8,), jnp.int32),
            pltpu.VMEM((_BLK, _BLK), jnp.float32),
            pltpu.VMEM((_BLK, _BLK), jnp.float32),
            pltpu.VMEM((_BLK, _BLK), jnp.float32),
            pltpu.SemaphoreType.DMA,
            pltpu.SemaphoreType.DMA,
            pltpu.SemaphoreType.DMA,
            pltpu.SemaphoreType.DMA,
            pltpu.SemaphoreType.DMA,
            pltpu.SemaphoreType.DMA,
            pltpu.SemaphoreType.DMA,
            pltpu.SemaphoreType.DMA,
            pltpu.SemaphoreType.DMA,
        ],
    )(modified_adj)
    return out
